# baseline (speedup 1.0000x reference)
import functools

import jax
import jax.numpy as jnp
from jax import lax
from jax.experimental import pallas as pl
from jax.experimental.pallas import tpu as pltpu
from jax.experimental.pallas import tpu_sc as plsc

N = 100000
NP = 102400
E = 1600000
EP = 1605632
EROWS = EP // 128
H = 64
NRL = NP // 2
ACC_ROWS = 51328
TRASH = 51200
BR = 2048
NBLK = NP // BR

_f32 = jnp.float32
_DN = (((1,), (1,)), ((), ()))

_mesh = plsc.VectorSubcoreMesh(core_axis_name="c", subcore_axis_name="s")
_sc_params = pltpu.CompilerParams(use_tc_tiling_on_sc=False)


@functools.partial(
    pl.kernel,
    out_type=jax.ShapeDtypeStruct((2, NP), _f32),
    mesh=_mesh,
    compiler_params=_sc_params,
    scratch_types=[
        pltpu.VMEM((1024,), jnp.int32),
        pltpu.VMEM((1024,), _f32),
        pltpu.VMEM((6400,), _f32),
        pltpu.VMEM_SHARED((NP,), _f32),
    ],
)
def _deg_kernel(dst_hbm, out_hbm, dstb, ones, zbuf, counts):
    c = lax.axis_index("c")
    s = lax.axis_index("s")

    def _fill(i, carry):
        zbuf[pl.ds(i * 16, 16)] = jnp.zeros((16,), _f32)
        return carry

    lax.fori_loop(0, 400, _fill, 0)

    def _fill1(i, carry):
        ones[pl.ds(i * 16, 16)] = jnp.ones((16,), _f32)
        return carry

    lax.fori_loop(0, 64, _fill1, 0)

    pltpu.sync_copy(zbuf, counts.at[pl.ds(s * 6400, 6400)])
    plsc.subcore_barrier()

    e0 = c * (EP // 2) + s * 50176

    def _chunk(t, carry):
        pltpu.sync_copy(dst_hbm.at[pl.ds(e0 + t * 1024, 1024)], dstb)
        pltpu.sync_copy(ones, counts.at[dstb], add=True)
        return carry

    lax.fori_loop(0, 49, _chunk, 0)
    plsc.subcore_barrier()
    pltpu.sync_copy(counts.at[pl.ds(s * 6400, 6400)],
                    out_hbm.at[c, pl.ds(s * 6400, 6400)])


@functools.partial(
    pl.kernel,
    out_type=(jax.ShapeDtypeStruct((2, NRL, 32), _f32),
              jax.ShapeDtypeStruct((2, NRL, 32), _f32)),
    mesh=_mesh,
    compiler_params=_sc_params,
    scratch_types=[
        pltpu.VMEM((2048,), jnp.int32),
        pltpu.VMEM((2048,), jnp.int32),
        pltpu.VMEM((512,), jnp.int32),
        pltpu.VMEM((512, 32), _f32),
        pltpu.VMEM((136, 32), _f32),
        pltpu.VMEM_SHARED((ACC_ROWS, 32), _f32),
        pltpu.SemaphoreType.DMA,
        pltpu.SemaphoreType.DMA,
    ],
)
def _prop_kernel(htA, htB, src_hbm, dst_hbm, outA, outB,
                 srcbig, dstbig, ldstb, rowsb, zbuf, acc, semG, semS):
    c = lax.axis_index("c")
    s = lax.axis_index("s")
    base = c * NRL

    def _zfill(i, carry):
        zbuf[i, pl.ds(0, 16)] = jnp.zeros((16,), _f32)
        zbuf[i, pl.ds(16, 16)] = jnp.zeros((16,), _f32)
        return carry

    lax.fori_loop(0, 136, _zfill, 0)

    e0 = s * 100352

    for p in range(2):
        tbl = htA if p == 0 else htB
        outp = outA if p == 0 else outB

        z0 = s * 3208
        for q in range(23):
            pltpu.sync_copy(zbuf, acc.at[pl.ds(z0 + q * 136, 136)])
        pltpu.sync_copy(zbuf.at[pl.ds(0, 80)], acc.at[pl.ds(z0 + 3128, 80)])
        plsc.subcore_barrier()

        def _chunk(t, carry):
            @pl.when(t >= 1)
            def _drain():
                pltpu.make_async_copy(rowsb, acc.at[ldstb], semS).wait()

            @pl.when(t % 4 == 0)
            def _refill():
                r = e0 + (t // 4) * 2048
                pltpu.sync_copy(src_hbm.at[pl.ds(r, 2048)], srcbig)
                pltpu.sync_copy(dst_hbm.at[pl.ds(r, 2048)], dstbig)

            blk = (t % 4) * 512
            gd = pltpu.async_copy(tbl.at[srcbig.at[pl.ds(blk, 512)]], rowsb,
                                  semG)
            for k in range(32):
                d = dstbig[pl.ds(blk + k * 16, 16)]
                ld = d - base
                m = (ld >= 0) & (ld < NRL)
                ldstb[pl.ds(k * 16, 16)] = jnp.where(m, ld, TRASH)
            gd.wait()
            pltpu.async_copy(rowsb, acc.at[ldstb], semS, add=True)
            return carry

        lax.fori_loop(0, 196, _chunk, 0)
        pltpu.make_async_copy(rowsb, acc.at[ldstb], semS).wait()
        plsc.subcore_barrier()
        wb = s * 3200
        pltpu.sync_copy(acc.at[pl.ds(wb, 3200)], outp.at[c, pl.ds(wb, 3200)])
        if p == 0:
            plsc.subcore_barrier()


def _dinv_body(c0_ref, c1_ref, out_ref):
    deg = c0_ref[...] + c1_ref[...] + 1.0
    idx = (lax.broadcasted_iota(jnp.int32, (800, 128), 0) * 128
           + lax.broadcasted_iota(jnp.int32, (800, 128), 1))
    out_ref[...] = jnp.where(idx < N, lax.rsqrt(deg), 0.0)


_dinv_call = pl.pallas_call(
    _dinv_body,
    out_shape=jax.ShapeDtypeStruct((800, 128), _f32),
)


def _pre0_body(x_ref, w_ref, b_ref, h_ref):
    h = lax.dot_general(x_ref[...], w_ref[...], _DN, preferred_element_type=_f32)
    h_ref[...] = jnp.maximum(h + b_ref[...], 0.0)


_pre0_call = pl.pallas_call(
    _pre0_body,
    grid=(NBLK,),
    in_specs=[
        pl.BlockSpec((BR, 128), lambda i: (i, 0)),
        pl.BlockSpec((H, 128), lambda i: (0, 0)),
        pl.BlockSpec((1, H), lambda i: (0, 0)),
    ],
    out_specs=pl.BlockSpec((BR, H), lambda i: (i, 0)),
    out_shape=jax.ShapeDtypeStruct((NP, H), _f32),
)


def _pre_body(h_ref, w_ref, dinv_ref, a_ref, b_ref):
    ht = lax.dot_general(h_ref[...], w_ref[...], _DN, preferred_element_type=_f32)
    htil = ht * dinv_ref[...]
    a_ref[...] = htil[:, :32]
    b_ref[...] = htil[:, 32:]


_pre_call = pl.pallas_call(
    _pre_body,
    grid=(NBLK,),
    in_specs=[
        pl.BlockSpec((BR, H), lambda i: (i, 0)),
        pl.BlockSpec((H, H), lambda i: (0, 0)),
        pl.BlockSpec((BR, 1), lambda i: (i, 0)),
    ],
    out_specs=(pl.BlockSpec((BR, 32), lambda i: (i, 0)),
               pl.BlockSpec((BR, 32), lambda i: (i, 0))),
    out_shape=(jax.ShapeDtypeStruct((NP, 32), _f32),
               jax.ShapeDtypeStruct((NP, 32), _f32)),
)


def _mid_body(aA_ref, aB_ref, tA_ref, tB_ref, hp_ref, dinv_ref,
              bc_ref, g_ref, be_ref, out_ref):
    acc = jnp.concatenate([aA_ref[...], aB_ref[...]], axis=1)
    til = jnp.concatenate([tA_ref[...], tB_ref[...]], axis=1)
    z = dinv_ref[...] * (acc + til) + bc_ref[...]
    mu = jnp.mean(z, axis=1, keepdims=True)
    zc = z - mu
    var = jnp.mean(zc * zc, axis=1, keepdims=True)
    y = zc * lax.rsqrt(var + 1e-5) * g_ref[...] + be_ref[...]
    out_ref[...] = jnp.maximum(y, 0.0) + hp_ref[...]


_mid_call = pl.pallas_call(
    _mid_body,
    grid=(NBLK,),
    in_specs=[
        pl.BlockSpec((BR, 32), lambda i: (i, 0)),
        pl.BlockSpec((BR, 32), lambda i: (i, 0)),
        pl.BlockSpec((BR, 32), lambda i: (i, 0)),
        pl.BlockSpec((BR, 32), lambda i: (i, 0)),
        pl.BlockSpec((BR, H), lambda i: (i, 0)),
        pl.BlockSpec((BR, 1), lambda i: (i, 0)),
        pl.BlockSpec((1, H), lambda i: (0, 0)),
        pl.BlockSpec((1, H), lambda i: (0, 0)),
        pl.BlockSpec((1, H), lambda i: (0, 0)),
    ],
    out_specs=pl.BlockSpec((BR, H), lambda i: (i, 0)),
    out_shape=jax.ShapeDtypeStruct((NP, H), _f32),
)


def _head_body(h_ref, w1_ref, b1_ref, w2_ref, b2_ref, out_ref):
    t = lax.dot_general(h_ref[...], w1_ref[...], _DN, preferred_element_type=_f32)
    t = jnp.maximum(t + b1_ref[...], 0.0)
    o = jnp.sum(t * w2_ref[...], axis=1, keepdims=True)
    out_ref[...] = jax.nn.sigmoid(o + b2_ref[0, 0])


_head_call = pl.pallas_call(
    _head_body,
    grid=(NBLK,),
    in_specs=[
        pl.BlockSpec((BR, H), lambda i: (i, 0)),
        pl.BlockSpec((32, H), lambda i: (0, 0)),
        pl.BlockSpec((1, 32), lambda i: (0, 0)),
        pl.BlockSpec((1, 32), lambda i: (0, 0)),
        pl.BlockSpec((1, 1), lambda i: (0, 0)),
    ],
    out_specs=pl.BlockSpec((BR, 1), lambda i: (i, 0)),
    out_shape=jax.ShapeDtypeStruct((NP, 1), _f32),
)


def kernel(x, edge_index, W_in, b_in, Wc1, bc1, g1, be1, Wc2, bc2, g2, be2,
           Wc3, bc3, g3, be3, Wo1, bo1, Wo2, bo2):
    src = edge_index[0]
    dst = edge_index[1]
    pad_src = jnp.zeros((EP - E,), dtype=src.dtype)
    pad_dst = jnp.full((EP - E,), N, dtype=dst.dtype)
    src2 = jnp.concatenate([src, pad_src])
    dst2 = jnp.concatenate([dst, pad_dst])
    xp = jnp.pad(x, ((0, NP - N), (0, 0)))

    counts = _deg_kernel(dst2)
    dinv = _dinv_call(counts[0].reshape(800, 128),
                      counts[1].reshape(800, 128)).reshape(NP, 1)

    h = _pre0_call(xp, W_in, b_in.reshape(1, H))
    for Wc, bc, g, be in ((Wc1, bc1, g1, be1), (Wc2, bc2, g2, be2),
                          (Wc3, bc3, g3, be3)):
        tA, tB = _pre_call(h, Wc, dinv)
        aA, aB = _prop_kernel(tA, tB, src2, dst2)
        h = _mid_call(aA.reshape(NP, 32), aB.reshape(NP, 32), tA, tB, h, dinv,
                      bc.reshape(1, H), g.reshape(1, H), be.reshape(1, H))

    out = _head_call(h, Wo1, bo1.reshape(1, 32), Wo2, bo2.reshape(1, 1))
    return out[:N]


# speedup vs baseline: 1.6334x; 1.6334x over previous
import functools

import jax
import jax.numpy as jnp
from jax import lax
from jax.experimental import pallas as pl
from jax.experimental.pallas import tpu as pltpu
from jax.experimental.pallas import tpu_sc as plsc

N = 100000
NP = 102400
E = 1600000
EP = 1605632
EROWS = EP // 128
H = 64
NRL = NP // 2
ACC_ROWS = 51328
TRASH = 51200
BR = 2048
NBLK = NP // BR

_f32 = jnp.float32
_DN = (((1,), (1,)), ((), ()))

_mesh = plsc.VectorSubcoreMesh(core_axis_name="c", subcore_axis_name="s")
_sc_params = pltpu.CompilerParams(use_tc_tiling_on_sc=False,
                                  needs_layout_passes=False)


_DEG_SCRATCH = [
    pltpu.VMEM((1024,), jnp.int32),
    pltpu.VMEM((1024,), _f32),
    pltpu.VMEM((6400,), _f32),
    pltpu.VMEM_SHARED((NP,), _f32),
]


def _deg_body(dst_hbm, out_hbm, dstb, ones, zbuf, counts):
    c = lax.axis_index("c")
    s = lax.axis_index("s")

    def _fill(i, carry):
        zbuf[pl.ds(i * 16, 16)] = jnp.zeros((16,), _f32)
        return carry

    lax.fori_loop(0, 400, _fill, 0)

    def _fill1(i, carry):
        ones[pl.ds(i * 16, 16)] = jnp.ones((16,), _f32)
        return carry

    lax.fori_loop(0, 64, _fill1, 0)

    pltpu.sync_copy(zbuf, counts.at[pl.ds(s * 6400, 6400)])
    plsc.subcore_barrier()

    e0 = c * (EP // 2) + s * 50176

    def _chunk(t, carry):
        pltpu.sync_copy(dst_hbm.at[pl.ds(e0 + t * 1024, 1024)], dstb)
        pltpu.sync_copy(ones, counts.at[dstb], add=True)
        return carry

    lax.fori_loop(0, 49, _chunk, 0)
    plsc.subcore_barrier()
    pltpu.sync_copy(counts.at[pl.ds(s * 6400, 6400)],
                    out_hbm.at[c, pl.ds(s * 6400, 6400)])


_PROP_SCRATCH = [
    pltpu.VMEM((512,), jnp.int32),
    pltpu.VMEM((512,), jnp.int32),
    pltpu.VMEM((2, 1024), jnp.int32),
    pltpu.VMEM((2, 1024), jnp.int32),
    pltpu.VMEM((512, 32), _f32),
    pltpu.VMEM((136, 32), _f32),
    pltpu.VMEM_SHARED((ACC_ROWS, 32), _f32),
    pltpu.SemaphoreType.DMA,
    pltpu.SemaphoreType.DMA,
]


def _prop_body(htA, htB, src_hbm, dst_hbm, outA, outB,
                 srcb, dstb, csrc, cldst, rowsb, zbuf, acc, semG, semS):
    c = lax.axis_index("c")
    s = lax.axis_index("s")
    base = c * NRL

    def _zfill(i, carry):
        zbuf[i, pl.ds(0, 16)] = jnp.zeros((16,), _f32)
        zbuf[i, pl.ds(16, 16)] = jnp.zeros((16,), _f32)
        return carry

    lax.fori_loop(0, 136, _zfill, 0)

    e0 = s * 100352

    for p in range(2):
        tbl = htA if p == 0 else htB
        outp = outA if p == 0 else outB

        z0 = s * 3208
        for q in range(23):
            pltpu.sync_copy(zbuf, acc.at[pl.ds(z0 + q * 136, 136)])
        pltpu.sync_copy(zbuf.at[pl.ds(0, 80)], acc.at[pl.ds(z0 + 3128, 80)])
        plsc.subcore_barrier()

        def _flush(args):
            ptr, phase, started = args
            nphase = 1 - phase

            @pl.when(started == 1)
            def _drain():
                pltpu.make_async_copy(
                    rowsb, acc.at[cldst.at[nphase, pl.ds(0, 512)]], semS
                ).wait()

            pltpu.async_copy(tbl.at[csrc.at[phase, pl.ds(0, 512)]], rowsb,
                             semG).wait()
            for g in range(32):
                csrc[nphase, pl.ds(g * 16, 16)] = \
                    csrc[phase, pl.ds(512 + g * 16, 16)]
                cldst[nphase, pl.ds(g * 16, 16)] = \
                    cldst[phase, pl.ds(512 + g * 16, 16)]
            pltpu.async_copy(rowsb, acc.at[cldst.at[phase, pl.ds(0, 512)]],
                             semS, add=True)
            return (ptr - 512, nphase, 1)

        def _chunk(t, carry):
            ptr, phase, started = carry
            e = e0 + t * 512
            pltpu.sync_copy(src_hbm.at[pl.ds(e, 512)], srcb)
            pltpu.sync_copy(dst_hbm.at[pl.ds(e, 512)], dstb)
            for k in range(32):
                sv = srcb[pl.ds(k * 16, 16)]
                d = dstb[pl.ds(k * 16, 16)]
                ld = d - base
                m = (ld >= 0) & (ld < NRL)
                plsc.store_compressed(csrc.at[phase].at[pl.ds(ptr, 16)], sv,
                                      mask=m)
                plsc.store_compressed(cldst.at[phase].at[pl.ds(ptr, 16)], ld,
                                      mask=m)
                ptr = ptr + plsc.all_reduce_population_count(m)[0]
            return lax.cond(ptr >= 512, _flush, lambda a: a,
                            (ptr, phase, started))

        ptr, phase, started = lax.fori_loop(0, 196, _chunk, (0, 0, 0))

        lane = lax.broadcasted_iota(jnp.int32, (16,), 0)
        for g in range(32):
            pos = lane + g * 16
            stale = pos >= ptr
            cs = csrc[phase, pl.ds(g * 16, 16)]
            cl = cldst[phase, pl.ds(g * 16, 16)]
            csrc[phase, pl.ds(g * 16, 16)] = jnp.where(stale, 0, cs)
            cldst[phase, pl.ds(g * 16, 16)] = jnp.where(stale, TRASH, cl)

        @pl.when(started == 1)
        def _drain_last():
            pltpu.make_async_copy(
                rowsb, acc.at[cldst.at[1 - phase, pl.ds(0, 512)]], semS
            ).wait()

        pltpu.async_copy(tbl.at[csrc.at[phase, pl.ds(0, 512)]], rowsb,
                         semG).wait()
        pltpu.sync_copy(rowsb, acc.at[cldst.at[phase, pl.ds(0, 512)]],
                        add=True)
        plsc.subcore_barrier()
        wb = s * 3200
        pltpu.sync_copy(acc.at[pl.ds(wb, 3200)], outp.at[c, pl.ds(wb, 3200)])
        if p == 0:
            plsc.subcore_barrier()


_deg_kernel = pl.kernel(
    _deg_body,
    out_type=jax.ShapeDtypeStruct((2, NP), _f32),
    mesh=_mesh,
    compiler_params=_sc_params,
    scratch_types=_DEG_SCRATCH,
)

_prop_kernel = pl.kernel(
    _prop_body,
    out_type=(jax.ShapeDtypeStruct((2, NRL, 32), _f32),
              jax.ShapeDtypeStruct((2, NRL, 32), _f32)),
    mesh=_mesh,
    compiler_params=_sc_params,
    scratch_types=_PROP_SCRATCH,
)


def _dinv_body(c0_ref, c1_ref, out_ref):
    deg = c0_ref[...] + c1_ref[...] + 1.0
    idx = (lax.broadcasted_iota(jnp.int32, (800, 128), 0) * 128
           + lax.broadcasted_iota(jnp.int32, (800, 128), 1))
    out_ref[...] = jnp.where(idx < N, lax.rsqrt(deg), 0.0)


_dinv_call = pl.pallas_call(
    _dinv_body,
    out_shape=jax.ShapeDtypeStruct((800, 128), _f32),
)


def _pre0_body(x_ref, w_ref, b_ref, h_ref):
    h = lax.dot_general(x_ref[...], w_ref[...], _DN, preferred_element_type=_f32)
    h_ref[...] = jnp.maximum(h + b_ref[...], 0.0)


_pre0_call = pl.pallas_call(
    _pre0_body,
    grid=(NBLK,),
    in_specs=[
        pl.BlockSpec((BR, 128), lambda i: (i, 0)),
        pl.BlockSpec((H, 128), lambda i: (0, 0)),
        pl.BlockSpec((1, H), lambda i: (0, 0)),
    ],
    out_specs=pl.BlockSpec((BR, H), lambda i: (i, 0)),
    out_shape=jax.ShapeDtypeStruct((NP, H), _f32),
)


def _pre_body(h_ref, w_ref, dinv_ref, a_ref, b_ref):
    ht = lax.dot_general(h_ref[...], w_ref[...], _DN, preferred_element_type=_f32)
    htil = ht * dinv_ref[...]
    a_ref[...] = htil[:, :32]
    b_ref[...] = htil[:, 32:]


_pre_call = pl.pallas_call(
    _pre_body,
    grid=(NBLK,),
    in_specs=[
        pl.BlockSpec((BR, H), lambda i: (i, 0)),
        pl.BlockSpec((H, H), lambda i: (0, 0)),
        pl.BlockSpec((BR, 1), lambda i: (i, 0)),
    ],
    out_specs=(pl.BlockSpec((BR, 32), lambda i: (i, 0)),
               pl.BlockSpec((BR, 32), lambda i: (i, 0))),
    out_shape=(jax.ShapeDtypeStruct((NP, 32), _f32),
               jax.ShapeDtypeStruct((NP, 32), _f32)),
)


def _mid_body(aA_ref, aB_ref, tA_ref, tB_ref, hp_ref, dinv_ref,
              bc_ref, g_ref, be_ref, out_ref):
    acc = jnp.concatenate([aA_ref[...], aB_ref[...]], axis=1)
    til = jnp.concatenate([tA_ref[...], tB_ref[...]], axis=1)
    z = dinv_ref[...] * (acc + til) + bc_ref[...]
    mu = jnp.mean(z, axis=1, keepdims=True)
    zc = z - mu
    var = jnp.mean(zc * zc, axis=1, keepdims=True)
    y = zc * lax.rsqrt(var + 1e-5) * g_ref[...] + be_ref[...]
    out_ref[...] = jnp.maximum(y, 0.0) + hp_ref[...]


_mid_call = pl.pallas_call(
    _mid_body,
    grid=(NBLK,),
    in_specs=[
        pl.BlockSpec((BR, 32), lambda i: (i, 0)),
        pl.BlockSpec((BR, 32), lambda i: (i, 0)),
        pl.BlockSpec((BR, 32), lambda i: (i, 0)),
        pl.BlockSpec((BR, 32), lambda i: (i, 0)),
        pl.BlockSpec((BR, H), lambda i: (i, 0)),
        pl.BlockSpec((BR, 1), lambda i: (i, 0)),
        pl.BlockSpec((1, H), lambda i: (0, 0)),
        pl.BlockSpec((1, H), lambda i: (0, 0)),
        pl.BlockSpec((1, H), lambda i: (0, 0)),
    ],
    out_specs=pl.BlockSpec((BR, H), lambda i: (i, 0)),
    out_shape=jax.ShapeDtypeStruct((NP, H), _f32),
)


def _head_body(h_ref, w1_ref, b1_ref, w2_ref, b2_ref, out_ref):
    t = lax.dot_general(h_ref[...], w1_ref[...], _DN, preferred_element_type=_f32)
    t = jnp.maximum(t + b1_ref[...], 0.0)
    o = jnp.sum(t * w2_ref[...], axis=1, keepdims=True)
    out_ref[...] = jax.nn.sigmoid(o + b2_ref[0, 0])


_head_call = pl.pallas_call(
    _head_body,
    grid=(NBLK,),
    in_specs=[
        pl.BlockSpec((BR, H), lambda i: (i, 0)),
        pl.BlockSpec((32, H), lambda i: (0, 0)),
        pl.BlockSpec((1, 32), lambda i: (0, 0)),
        pl.BlockSpec((1, 32), lambda i: (0, 0)),
        pl.BlockSpec((1, 1), lambda i: (0, 0)),
    ],
    out_specs=pl.BlockSpec((BR, 1), lambda i: (i, 0)),
    out_shape=jax.ShapeDtypeStruct((NP, 1), _f32),
)


def kernel(x, edge_index, W_in, b_in, Wc1, bc1, g1, be1, Wc2, bc2, g2, be2,
           Wc3, bc3, g3, be3, Wo1, bo1, Wo2, bo2):
    src = edge_index[0]
    dst = edge_index[1]
    pad_src = jnp.zeros((EP - E,), dtype=src.dtype)
    pad_dst = jnp.full((EP - E,), N, dtype=dst.dtype)
    src2 = jnp.concatenate([src, pad_src])
    dst2 = jnp.concatenate([dst, pad_dst])
    xp = jnp.pad(x, ((0, NP - N), (0, 0)))

    counts = _deg_kernel(dst2)
    dinv = _dinv_call(counts[0].reshape(800, 128),
                      counts[1].reshape(800, 128)).reshape(NP, 1)

    h = _pre0_call(xp, W_in, b_in.reshape(1, H))
    for Wc, bc, g, be in ((Wc1, bc1, g1, be1), (Wc2, bc2, g2, be2),
                          (Wc3, bc3, g3, be3)):
        tA, tB = _pre_call(h, Wc, dinv)
        aA, aB = _prop_kernel(tA, tB, src2, dst2)
        h = _mid_call(aA.reshape(NP, 32), aB.reshape(NP, 32), tA, tB, h, dinv,
                      bc.reshape(1, H), g.reshape(1, H), be.reshape(1, H))

    out = _head_call(h, Wo1, bo1.reshape(1, 32), Wo2, bo2.reshape(1, 1))
    return out[:N]
